# baseline (speedup 1.0000x reference)
import jax
import jax.numpy as jnp
from jax.experimental import pallas as pl
from jax.experimental.pallas import tpu as pltpu

_DMODEL = 1024
_NE = 8
_ES = 128
_NT = 2048
_BLK = 512
_NBLK = _NT // _BLK


def _moe_body(x_ref, keys_ref, values_ref, es_ref, out_ref, reg_ref,
              kmat_ref, vmat_ref, s_ref):
    i = pl.program_id(0)

    @pl.when(i == 0)
    def _():
        s_ref[...] = jnp.zeros_like(s_ref)
        for e in range(_NE):
            kmat_ref[:, e * _ES:(e + 1) * _ES] = keys_ref[e]
            vmat_ref[e * _ES:(e + 1) * _ES, :] = values_ref[e]

    x = x_ref[...]
    sel_raw = jax.lax.dot_general(
        x, es_ref[...], (((1,), (1,)), ((), ())),
        preferred_element_type=jnp.float32)

    p = jnp.exp(sel_raw)
    p = p / jnp.sum(p, axis=-1, keepdims=True)
    s_ref[...] += jnp.sum(p, axis=0, keepdims=True)

    cols = jax.lax.broadcasted_iota(jnp.int32, sel_raw.shape, 1)
    idx1 = jnp.argmax(sel_raw, axis=-1)[:, None]
    v1 = jnp.max(sel_raw, axis=-1, keepdims=True)
    masked = jnp.where(cols == idx1, -jnp.inf, sel_raw)
    idx2 = jnp.argmax(masked, axis=-1)[:, None]
    v2 = jnp.max(masked, axis=-1, keepdims=True)
    g1 = jax.nn.sigmoid(v1)
    g2 = jax.nn.sigmoid(v2)
    half = _NE * _ES // 2
    acc = None
    for j in range(2):
        hj = jax.lax.dot_general(
            x, kmat_ref[:, j * half:(j + 1) * half],
            (((1,), (0,)), ((), ())),
            preferred_element_type=jnp.float32)
        ecol = (jax.lax.broadcasted_iota(jnp.int32, hj.shape, 1)
                + j * half) >> 7
        wj = (jnp.where(ecol == idx1, g1, 0.0)
              + jnp.where(ecol == idx2, g2, 0.0))
        hj = jnp.maximum(hj, 0.0) * wj
        oj = jax.lax.dot_general(
            hj, vmat_ref[j * half:(j + 1) * half, :],
            (((1,), (0,)), ((), ())),
            preferred_element_type=jnp.float32)
        acc = oj if acc is None else acc + oj
    out_ref[...] = acc

    @pl.when(i == _NBLK - 1)
    def _():
        s = s_ref[...]
        lm = jnp.log(s) - jnp.log(float(_NT))
        reg_ref[...] = jnp.sum(lm * (s / float(_NT)), axis=1, keepdims=True)


def kernel(x, keys, values, expert_sel):
    out, reg = pl.pallas_call(
        _moe_body,
        grid=(_NBLK,),
        in_specs=[
            pl.BlockSpec((_BLK, _DMODEL), lambda i: (i, 0)),
            pl.BlockSpec((_NE, _DMODEL, _ES), lambda i: (0, 0, 0)),
            pl.BlockSpec((_NE, _ES, _DMODEL), lambda i: (0, 0, 0)),
            pl.BlockSpec((_NE, _DMODEL), lambda i: (0, 0)),
        ],
        out_specs=[
            pl.BlockSpec((_BLK, _DMODEL), lambda i: (i, 0)),
            pl.BlockSpec((1, 1), lambda i: (0, 0)),
        ],
        out_shape=[
            jax.ShapeDtypeStruct((_NT, _DMODEL), jnp.float32),
            jax.ShapeDtypeStruct((1, 1), jnp.float32),
        ],
        scratch_shapes=[
            pltpu.VMEM((_DMODEL, _NE * _ES), jnp.float32),
            pltpu.VMEM((_NE * _ES, _DMODEL), jnp.float32),
            pltpu.VMEM((1, _NE), jnp.float32),
        ],
    )(x, keys, values, expert_sel)
    return out, reg[0, 0]


# speedup vs baseline: 1.2349x; 1.2349x over previous
import jax
import jax.numpy as jnp
from jax.experimental import pallas as pl
from jax.experimental.pallas import tpu as pltpu

_DMODEL = 1024
_NE = 8
_ES = 128
_NT = 2048
_BLK = 512
_NBLK = _NT // _BLK


def _moe_body(x_ref, keys_ref, values_ref, es_ref, out_ref, reg_ref,
              kmat_ref, vmat_ref, s_ref):
    i = pl.program_id(0)

    @pl.when(i == 0)
    def _():
        s_ref[...] = jnp.zeros_like(s_ref)
        for e in range(_NE):
            kmat_ref[:, e * _ES:(e + 1) * _ES] = keys_ref[e]
            vmat_ref[e * _ES:(e + 1) * _ES, :] = values_ref[e]

    x = x_ref[...]
    sel_raw = jax.lax.dot_general(
        x, es_ref[...], (((1,), (1,)), ((), ())),
        preferred_element_type=jnp.float32)

    p = jnp.exp(sel_raw)
    p = p / jnp.sum(p, axis=-1, keepdims=True)
    s_ref[...] += jnp.sum(p, axis=0, keepdims=True)

    cols = jax.lax.broadcasted_iota(jnp.int32, sel_raw.shape, 1)
    idx1 = jnp.argmax(sel_raw, axis=-1)[:, None]
    v1 = jnp.max(sel_raw, axis=-1, keepdims=True)
    masked = jnp.where(cols == idx1, -jnp.inf, sel_raw)
    idx2 = jnp.argmax(masked, axis=-1)[:, None]
    v2 = jnp.max(masked, axis=-1, keepdims=True)
    g1 = jax.nn.sigmoid(v1)
    g2 = jax.nn.sigmoid(v2)
    h = jax.lax.dot_general(
        x, kmat_ref[...], (((1,), (0,)), ((), ())),
        preferred_element_type=jnp.float32)
    ecol = jax.lax.broadcasted_iota(jnp.int32, h.shape, 1) >> 7
    w = (jnp.where(ecol == idx1, g1, 0.0)
         + jnp.where(ecol == idx2, g2, 0.0))
    h = jnp.maximum(h, 0.0) * w
    out_ref[...] = jax.lax.dot_general(
        h, vmat_ref[...], (((1,), (0,)), ((), ())),
        preferred_element_type=jnp.float32)

    @pl.when(i == _NBLK - 1)
    def _():
        s = s_ref[...]
        lm = jnp.log(s) - jnp.log(float(_NT))
        reg_ref[...] = jnp.sum(lm * (s / float(_NT)), axis=1, keepdims=True)


def kernel(x, keys, values, expert_sel):
    out, reg = pl.pallas_call(
        _moe_body,
        grid=(_NBLK,),
        in_specs=[
            pl.BlockSpec((_BLK, _DMODEL), lambda i: (i, 0)),
            pl.BlockSpec((_NE, _DMODEL, _ES), lambda i: (0, 0, 0)),
            pl.BlockSpec((_NE, _ES, _DMODEL), lambda i: (0, 0, 0)),
            pl.BlockSpec((_NE, _DMODEL), lambda i: (0, 0)),
        ],
        out_specs=[
            pl.BlockSpec((_BLK, _DMODEL), lambda i: (i, 0)),
            pl.BlockSpec((1, 1), lambda i: (0, 0)),
        ],
        out_shape=[
            jax.ShapeDtypeStruct((_NT, _DMODEL), jnp.float32),
            jax.ShapeDtypeStruct((1, 1), jnp.float32),
        ],
        scratch_shapes=[
            pltpu.VMEM((_DMODEL, _NE * _ES), jnp.float32),
            pltpu.VMEM((_NE * _ES, _DMODEL), jnp.float32),
            pltpu.VMEM((1, _NE), jnp.float32),
        ],
    )(x, keys, values, expert_sel)
    return out, reg[0, 0]
